# baseline (speedup 1.0000x reference)
import functools

import jax
import jax.numpy as jnp
from jax import lax
from jax.experimental import pallas as pl
from jax.experimental.pallas import tpu as pltpu
from jax.experimental.pallas import tpu_sc as plsc

_NC = 2
_NS = 16
_NW = _NC * _NS
_ND = 8


@functools.partial(jax.jit, static_argnums=(2, 3))
def _sc_gather(tbl, sidx, n_sub, dc):
    s_per_w = n_sub // _NW
    mesh = plsc.VectorSubcoreMesh(
        core_axis_name="c", subcore_axis_name="s",
        num_cores=_NC, num_subcores=_NS)

    @functools.partial(
        pl.kernel,
        out_type=jax.ShapeDtypeStruct((n_sub, dc), jnp.float32),
        mesh=mesh,
        scratch_types=[
            pltpu.VMEM((s_per_w,), jnp.int32),
            pltpu.VMEM((_ND, dc), jnp.float32),
            pltpu.SemaphoreType.DMA,
        ],
    )
    def k(tbl_hbm, sidx_hbm, out_hbm, idx_v, row_v, gsem):
        wid = lax.axis_index("s") * _NC + lax.axis_index("c")
        sbase = wid * s_per_w
        pltpu.sync_copy(sidx_hbm.at[pl.ds(sbase, s_per_w)], idx_v)

        @pl.loop(0, s_per_w // _ND)
        def _(i):
            pltpu.async_copy(
                tbl_hbm.at[idx_v.at[pl.ds(i * _ND, _ND)]], row_v, gsem
            ).wait()
            pltpu.sync_copy(row_v, out_hbm.at[pl.ds(sbase + i * _ND, _ND)])

    return k(tbl, sidx)


def kernel(prefix, embedding_table):
    V, D = embedding_table.shape
    B = prefix.size
    dc = D // _ND
    idx = prefix.reshape(-1).astype(jnp.int32)
    sidx = (idx[:, None] * _ND + jnp.arange(_ND, dtype=jnp.int32)).reshape(-1)
    tbl = embedding_table.reshape(V * _ND, dc)
    out = _sc_gather(tbl, sidx, B * _ND, dc)
    return out.reshape(*prefix.shape, D)


# speedup vs baseline: 1.0368x; 1.0368x over previous
import functools

import jax
import jax.numpy as jnp
from jax import lax
from jax.experimental import pallas as pl
from jax.experimental.pallas import tpu as pltpu
from jax.experimental.pallas import tpu_sc as plsc

_NC = 2
_NS = 16
_NW = _NC * _NS
_ND = 8


@functools.partial(jax.jit, static_argnums=(2, 3))
def _sc_gather(tbl, sidx, n_sub, dc):
    s_per_w = n_sub // _NW
    mesh = plsc.VectorSubcoreMesh(
        core_axis_name="c", subcore_axis_name="s",
        num_cores=_NC, num_subcores=_NS)

    @functools.partial(
        pl.kernel,
        out_type=jax.ShapeDtypeStruct((n_sub, dc), jnp.float32),
        mesh=mesh,
        scratch_types=[
            pltpu.VMEM((s_per_w,), jnp.int32),
            pltpu.VMEM((_ND, dc), jnp.float32),
            pltpu.VMEM((_ND, dc), jnp.float32),
            pltpu.SemaphoreType.DMA,
            pltpu.SemaphoreType.DMA,
            pltpu.SemaphoreType.DMA,
            pltpu.SemaphoreType.DMA,
        ],
    )
    def k(tbl_hbm, sidx_hbm, out_hbm, idx_v, row0, row1, g0, g1, s0, s1):
        wid = lax.axis_index("s") * _NC + lax.axis_index("c")
        sbase = wid * s_per_w
        n_rows = s_per_w // _ND
        pltpu.sync_copy(sidx_hbm.at[pl.ds(sbase, s_per_w)], idx_v)

        def gather(r, buf, sem):
            return pltpu.make_async_copy(
                tbl_hbm.at[idx_v.at[pl.ds(r * _ND, _ND)]], buf, sem)

        def scatter(r, buf, sem):
            return pltpu.make_async_copy(
                buf, out_hbm.at[pl.ds(sbase + r * _ND, _ND)], sem)

        gather(0, row0, g0).start()
        gather(0, row0, g0).wait()
        scatter(0, row0, s0).start()
        gather(1, row1, g1).start()

        @pl.loop(0, (n_rows - 2) // 2)
        def _(j):
            a = 2 * j + 1
            gather(a, row1, g1).wait()
            scatter(a, row1, s1).start()
            scatter(a - 1, row0, s0).wait()
            gather(a + 1, row0, g0).start()
            gather(a + 1, row0, g0).wait()
            scatter(a + 1, row0, s0).start()
            scatter(a, row1, s1).wait()
            gather(a + 2, row1, g1).start()

        last = n_rows - 1
        gather(last, row1, g1).wait()
        scatter(last, row1, s1).start()
        scatter(last - 1, row0, s0).wait()
        scatter(last, row1, s1).wait()

    return k(tbl, sidx)


def kernel(prefix, embedding_table):
    V, D = embedding_table.shape
    B = prefix.size
    dc = D // _ND
    idx = prefix.reshape(-1).astype(jnp.int32)
    sidx = (idx[:, None] * _ND + jnp.arange(_ND, dtype=jnp.int32)).reshape(-1)
    tbl = embedding_table.reshape(V * _ND, dc)
    out = _sc_gather(tbl, sidx, B * _ND, dc)
    return out.reshape(*prefix.shape, D)
